# baseline (speedup 1.0000x reference)
import functools

import jax
import jax.numpy as jnp
from jax import lax
from jax.experimental import pallas as pl
from jax.experimental.pallas import tpu as pltpu
from jax.experimental.pallas import tpu_sc as plsc

N = 10000
E = 320000
NC = 2
NS = 16
NW = NC * NS
EPW = E // NW
K = 80
UNROLL = 4
NB = 3
NCHUNK = EPW // K
SB = 5
CPW = NCHUNK
RPT = N // NS


def _sc_edge_pass(H, FW):
  D = 16
  EL0 = H * D
  mesh = plsc.VectorSubcoreMesh(core_axis_name="c", subcore_axis_name="s")

  @functools.partial(
      pl.kernel,
      out_type=jax.ShapeDtypeStruct((NC, N, FW), jnp.float32),
      mesh=mesh,
      compiler_params=pltpu.CompilerParams(use_tc_tiling_on_sc=False),
      scratch_types=[
          pltpu.VMEM_SHARED((N, FW), jnp.float32),
          pltpu.VMEM((2, SB, K), jnp.int32),
          pltpu.VMEM((2, SB, K), jnp.int32),
          pltpu.VMEM((NB, K, FW), jnp.float32),
          pltpu.VMEM((NB, K, 16), jnp.float32),
          pltpu.VMEM((16,), jnp.float32),
          pltpu.SemaphoreType.DMA((NB,)),
          pltpu.SemaphoreType.DMA((NB,)),
          pltpu.SemaphoreType.DMA((NB,)),
      ],
  )
  def edge_kernel(featx, er16, srcs, dsts, cvec, out, acc, sblk, dblk,
                  rows, erb, cbuf, semg, seme, sems):
    cid = lax.axis_index("c")
    sid = lax.axis_index("s")
    wid = sid * NC + cid

    zero = jnp.zeros((16,), jnp.float32)

    def zrow(r, carry):
      for j in range(FW // 16):
        rows[0, r, pl.ds(j * 16, 16)] = zero
      return carry

    lax.fori_loop(0, K, zrow, 0)
    for j in range(RPT // K):
      pltpu.sync_copy(rows.at[0], acc.at[pl.ds(sid * RPT + j * K, K)])
    rem = RPT % K
    if rem:
      pltpu.sync_copy(rows.at[0, pl.ds(0, rem)],
                      acc.at[pl.ds(sid * RPT + (RPT // K) * K, rem)])
    pltpu.sync_copy(cvec, cbuf)
    plsc.subcore_barrier()

    cv = cbuf[...]
    mask = lax.iota(jnp.int32, 16) < H
    crow0 = wid * CPW

    def load_block(nb):
      bb = lax.rem(nb, 2)
      pltpu.sync_copy(srcs.at[pl.ds(crow0 + nb * SB, SB)], sblk.at[bb])
      pltpu.sync_copy(dsts.at[pl.ds(crow0 + nb * SB, SB)], dblk.at[bb])

    def _gather_descs(ci):
      b = lax.rem(ci, NB)
      bb = lax.rem(ci // SB, 2)
      r = lax.rem(ci, SB)
      return (
          pltpu.make_async_copy(
              featx.at[sblk.at[bb, r]], rows.at[b], semg.at[b]),
          pltpu.make_async_copy(
              er16.at[dblk.at[bb, r]], erb.at[b], seme.at[b]),
      )

    def start_gather(ci):
      for d in _gather_descs(ci):
        d.start()

    def wait_gather(ci):
      for d in _gather_descs(ci):
        d.wait()

    def _scatter_desc(ci):
      b = lax.rem(ci, NB)
      bb = lax.rem(ci // SB, 2)
      r = lax.rem(ci, SB)
      return pltpu.make_async_copy(
          rows.at[b], acc.at[dblk.at[bb, r]], sems.at[b])

    def process(ci):
      b = lax.rem(ci, NB)
      rb = rows.at[b]
      eb = erb.at[b]

      def edge(e):
        elv = rb[e, pl.ds(EL0, 16)]
        erv = eb[e]
        t = elv + erv
        t = jnp.maximum(t, 0.2 * t)
        ee = jnp.exp(t - cv)
        for h in range(H):
          idx = jnp.full((16,), h, jnp.int32)
          bc = jnp.take_along_axis(ee, idx, axis=0, mode="promise_in_bounds")
          rb[e, pl.ds(h * D, 16)] = rb[e, pl.ds(h * D, 16)] * bc
        rb[e, pl.ds(EL0, 16)] = jnp.where(mask, ee, 0.0)

      def edgeu(i, c):
        for sub in range(UNROLL):
          edge(i * UNROLL + sub)
        return c

      lax.fori_loop(0, K // UNROLL, edgeu, 0)

    load_block(0)
    start_gather(0)

    def step(i, carry):
      @pl.when(i + 1 < NCHUNK)
      def _():
        @pl.when(lax.rem(i + 1, SB) == 0)
        def _():
          load_block((i + 1) // SB)

        @pl.when(i >= 2)
        def _():
          _scatter_desc(i - 2).wait()

        start_gather(i + 1)

      wait_gather(i)
      process(i)
      _scatter_desc(i).start(add=True)
      return carry

    lax.fori_loop(0, NCHUNK, step, 0)
    _scatter_desc(NCHUNK - 2).wait()
    _scatter_desc(NCHUNK - 1).wait()
    plsc.subcore_barrier()
    r0 = sid * RPT
    pltpu.sync_copy(acc.at[pl.ds(r0, RPT)], out.at[cid, pl.ds(r0, RPT)])

  return edge_kernel


_sc_l01 = _sc_edge_pass(8, 144)
_sc_l2 = _sc_edge_pass(1, 32)

BN = 2000
GRID = N // BN


def _full(shape):
  return pl.BlockSpec(shape, lambda i: (0,) * len(shape))


def _blk(w):
  return pl.BlockSpec((BN, w), lambda i: (i, 0))


def _tc_pre(x, W, Ablk, Bblk, FW, H):
  HD = H * 16
  pad = FW - HD - H

  def body(x_ref, w_ref, a_ref, b_ref, fx_ref, el_ref, er_ref):
    feat = jnp.dot(x_ref[...], w_ref[...], preferred_element_type=jnp.float32)
    el = jnp.dot(feat, a_ref[...], preferred_element_type=jnp.float32)
    er = jnp.dot(feat, b_ref[...], preferred_element_type=jnp.float32)
    z = jnp.zeros((BN, pad), jnp.float32)
    fx_ref[...] = jnp.concatenate([feat, el, z], axis=1)
    el_ref[...] = el
    er_ref[...] = jnp.concatenate(
        [er, jnp.zeros((BN, 16 - H), jnp.float32)], axis=1)

  return pl.pallas_call(
      body,
      grid=(GRID,),
      in_specs=[_blk(x.shape[1]), _full(W.shape), _full(Ablk.shape),
                _full(Bblk.shape)],
      out_specs=[_blk(FW), _blk(H), _blk(16)],
      out_shape=[
          jax.ShapeDtypeStruct((N, FW), jnp.float32),
          jax.ShapeDtypeStruct((N, H), jnp.float32),
          jax.ShapeDtypeStruct((N, 16), jnp.float32),
      ],
  )(x, W, Ablk, Bblk)


def _tc_mid(p0, p1, R, bvec, W, Ablk, Bblk, FW, H):
  HD = H * 16
  pad = FW - HD - H

  def body(p0_ref, p1_ref, r_ref, b_ref, w_ref, a_ref, bb_ref,
           fx_ref, el_ref, er_ref):
    acc = p0_ref[...] + p1_ref[...]
    esum = acc[:, 128:136]
    recip = jnp.where(esum > 0, 1.0 / esum, 0.0)
    scale = jnp.dot(recip, r_ref[...], preferred_element_type=jnp.float32)
    h = jnp.maximum(acc[:, 0:128] * scale + b_ref[...], 0.0)
    feat = jnp.dot(h, w_ref[...], preferred_element_type=jnp.float32)
    el = jnp.dot(feat, a_ref[...], preferred_element_type=jnp.float32)
    er = jnp.dot(feat, bb_ref[...], preferred_element_type=jnp.float32)
    z = jnp.zeros((BN, pad), jnp.float32)
    fx_ref[...] = jnp.concatenate([feat, el, z], axis=1)
    el_ref[...] = el
    er_ref[...] = jnp.concatenate(
        [er, jnp.zeros((BN, 16 - H), jnp.float32)], axis=1)

  return pl.pallas_call(
      body,
      grid=(GRID,),
      in_specs=[_blk(144), _blk(144), _full(R.shape), _full(bvec.shape),
                _full(W.shape), _full(Ablk.shape), _full(Bblk.shape)],
      out_specs=[_blk(FW), _blk(H), _blk(16)],
      out_shape=[
          jax.ShapeDtypeStruct((N, FW), jnp.float32),
          jax.ShapeDtypeStruct((N, H), jnp.float32),
          jax.ShapeDtypeStruct((N, 16), jnp.float32),
      ],
  )(p0, p1, R, bvec, W, Ablk, Bblk)


def _tc_final(q0, q1, b2vec):
  def body(q0_ref, q1_ref, b_ref, o_ref):
    acc = q0_ref[...] + q1_ref[...]
    esum = acc[:, 16:17]
    recip = jnp.where(esum > 0, 1.0 / esum, 0.0)
    o_ref[...] = acc[:, 0:16] * recip + b_ref[...]

  return pl.pallas_call(
      body,
      grid=(GRID,),
      in_specs=[_blk(32), _blk(32), _full(b2vec.shape)],
      out_specs=_blk(16),
      out_shape=jax.ShapeDtypeStruct((N, 16), jnp.float32),
  )(q0, q1, b2vec)


def _head_proj(a):
  H = a.shape[0]
  return (a[:, :, None] * jnp.eye(H, dtype=a.dtype)[:, None, :]).reshape(
      H * 16, H)


def _cvec(el, er16, H):
  c = jnp.maximum(jnp.max(el, axis=0) + jnp.max(er16[:, :H], axis=0), 0.0)
  return jnp.tile(c, 16 // H)


@jax.jit
def _run(inputs, edge_index, W0, al0, ar0, b0, W1, al1, ar1, b1,
         W2, al2, ar2, b2):
  src = edge_index[0].reshape(E // K, K)
  dst = edge_index[1].reshape(E // K, K)
  R = jnp.repeat(jnp.eye(8, dtype=jnp.float32), 16, axis=1)

  fx0, el0, er0 = _tc_pre(inputs, W0, _head_proj(al0), _head_proj(ar0), 144, 8)
  p = _sc_l01(fx0, er0, src, dst, _cvec(el0, er0, 8))

  fx1, el1, er1 = _tc_mid(p[0], p[1], R, b0.reshape(1, 128), W1,
                          _head_proj(al1), _head_proj(ar1), 144, 8)
  p = _sc_l01(fx1, er1, src, dst, _cvec(el1, er1, 8))

  fx2, el2, er2 = _tc_mid(p[0], p[1], R, b1.reshape(1, 128), W2,
                          _head_proj(al2), _head_proj(ar2), 32, 1)
  q = _sc_l2(fx2, er2, src, dst, _cvec(el2, er2, 1))

  return _tc_final(q[0], q[1], b2.reshape(1, 16))


def kernel(inputs, edge_index, W0, al0, ar0, b0, W1, al1, ar1, b1,
           W2, al2, ar2, b2):
  return _run(inputs, edge_index, W0, al0, ar0, b0, W1, al1, ar1, b1,
              W2, al2, ar2, b2)


# speedup vs baseline: 1.4356x; 1.4356x over previous
import functools

import jax
import jax.numpy as jnp
from jax import lax
from jax.experimental import pallas as pl
from jax.experimental.pallas import tpu as pltpu
from jax.experimental.pallas import tpu_sc as plsc

N = 10000
E = 320000
NC = 2
NS = 16
NW = NC * NS
EPW = E // NW
K = 80
UNROLL = 4
NCHUNK = EPW // K
SB = 5
CPW = NCHUNK
RPT = N // NS
RZB = 25


def _sc_edge_pass(H, FW):
  D = 16
  EL0 = H * D
  mesh = plsc.VectorSubcoreMesh(core_axis_name="c", subcore_axis_name="s")

  @functools.partial(
      pl.kernel,
      out_type=jax.ShapeDtypeStruct((NC, N, FW), jnp.float32),
      mesh=mesh,
      compiler_params=pltpu.CompilerParams(use_tc_tiling_on_sc=False),
      scratch_types=[
          pltpu.VMEM_SHARED((N, FW), jnp.float32),
          pltpu.VMEM((2, SB, K), jnp.int32),
          pltpu.VMEM((2, SB, K), jnp.int32),
          pltpu.VMEM((2, K, FW), jnp.float32),
          pltpu.VMEM((2, K, 16), jnp.float32),
          pltpu.VMEM((16,), jnp.float32),
          pltpu.VMEM((RZB, FW), jnp.float32),
          pltpu.SemaphoreType.DMA,
          pltpu.SemaphoreType.DMA,
          pltpu.SemaphoreType.DMA,
          pltpu.SemaphoreType.DMA,
      ],
  )
  def edge_kernel(featx, er16, srcs, dsts, cvec, out, acc, sidx, didx,
                  rows, erb, cbuf, zbuf, semr0, semr1, seme0, seme1):
    cid = lax.axis_index("c")
    sid = lax.axis_index("s")
    wid = sid * NC + cid
    semr = (semr0, semr1)
    seme = (seme0, seme1)

    zero = jnp.zeros((16,), jnp.float32)

    def zrow(r, carry):
      for j in range(FW // 16):
        zbuf[r, pl.ds(j * 16, 16)] = zero
      return carry

    lax.fori_loop(0, RZB, zrow, 0)
    for j in range(RPT // RZB):
      pltpu.sync_copy(zbuf, acc.at[pl.ds(sid * RPT + j * RZB, RZB)])
    pltpu.sync_copy(cvec, cbuf)
    plsc.subcore_barrier()

    cv = cbuf[...]
    mask = lax.iota(jnp.int32, 16) < H
    crow0 = wid * CPW

    def _idx_rows(ci):
      bb = lax.rem(ci // SB, 2)
      r = lax.rem(ci, SB)
      return sidx.at[bb, r], didx.at[bb, r]

    def start_gather(ci, b):
      @pl.when(lax.rem(ci, SB) == 0)
      def _():
        nb = ci // SB
        bb = lax.rem(nb, 2)
        pltpu.sync_copy(srcs.at[pl.ds(crow0 + nb * SB, SB)], sidx.at[bb])
        pltpu.sync_copy(dsts.at[pl.ds(crow0 + nb * SB, SB)], didx.at[bb])

      sr, dr = _idx_rows(ci)
      pltpu.make_async_copy(featx.at[sr], rows.at[b], semr[b]).start()
      pltpu.make_async_copy(er16.at[dr], erb.at[b], seme[b]).start()

    def wait_gather(ci, b):
      sr, dr = _idx_rows(ci)
      pltpu.make_async_copy(featx.at[sr], rows.at[b], semr[b]).wait()
      pltpu.make_async_copy(er16.at[dr], erb.at[b], seme[b]).wait()

    def process(ci, b):
      rb = rows.at[b]
      eb = erb.at[b]

      def edge(e):
        elv = rb[e, pl.ds(EL0, 16)]
        erv = eb[e]
        t = elv + erv
        t = jnp.maximum(t, 0.2 * t)
        ee = jnp.exp(t - cv)
        for h in range(H):
          idx = jnp.full((16,), h, jnp.int32)
          bc = jnp.take_along_axis(ee, idx, axis=0, mode="promise_in_bounds")
          rb[e, pl.ds(h * D, 16)] = rb[e, pl.ds(h * D, 16)] * bc
        rb[e, pl.ds(EL0, 16)] = jnp.where(mask, ee, 0.0)

      def edge4(i, c):
        for sub in range(UNROLL):
          edge(i * UNROLL + sub)
        return c

      lax.fori_loop(0, K // UNROLL, edge4, 0)
      _, dr = _idx_rows(ci)
      pltpu.sync_copy(rb, acc.at[dr], add=True)

    start_gather(0, 0)

    def pair(it, carry):
      c0 = it * 2
      start_gather(c0 + 1, 1)
      wait_gather(c0, 0)
      process(c0, 0)
      start_gather(c0 + 2, 0)
      wait_gather(c0 + 1, 1)
      process(c0 + 1, 1)
      return carry

    lax.fori_loop(0, (NCHUNK - 1) // 2, pair, 0)
    wait_gather(NCHUNK - 1, 0)
    process(NCHUNK - 1, 0)
    plsc.subcore_barrier()
    r0 = sid * RPT
    pltpu.sync_copy(acc.at[pl.ds(r0, RPT)], out.at[cid, pl.ds(r0, RPT)])

  return edge_kernel


_sc_l01 = _sc_edge_pass(8, 144)
_sc_l2 = _sc_edge_pass(1, 32)

BN = 2000
GRID = N // BN


def _full(shape):
  return pl.BlockSpec(shape, lambda i: (0,) * len(shape))


def _blk(w):
  return pl.BlockSpec((BN, w), lambda i: (i, 0))


def _tc_pre(x, W, Ablk, Bblk, FW, H):
  HD = H * 16
  pad = FW - HD - H

  def body(x_ref, w_ref, a_ref, b_ref, fx_ref, el_ref, er_ref):
    feat = jnp.dot(x_ref[...], w_ref[...], preferred_element_type=jnp.float32)
    el = jnp.dot(feat, a_ref[...], preferred_element_type=jnp.float32)
    er = jnp.dot(feat, b_ref[...], preferred_element_type=jnp.float32)
    z = jnp.zeros((BN, pad), jnp.float32)
    fx_ref[...] = jnp.concatenate([feat, el, z], axis=1)
    el_ref[...] = el
    er_ref[...] = jnp.concatenate(
        [er, jnp.zeros((BN, 16 - H), jnp.float32)], axis=1)

  return pl.pallas_call(
      body,
      grid=(GRID,),
      in_specs=[_blk(x.shape[1]), _full(W.shape), _full(Ablk.shape),
                _full(Bblk.shape)],
      out_specs=[_blk(FW), _blk(H), _blk(16)],
      out_shape=[
          jax.ShapeDtypeStruct((N, FW), jnp.float32),
          jax.ShapeDtypeStruct((N, H), jnp.float32),
          jax.ShapeDtypeStruct((N, 16), jnp.float32),
      ],
  )(x, W, Ablk, Bblk)


def _tc_mid(p0, p1, R, bvec, W, Ablk, Bblk, FW, H):
  HD = H * 16
  pad = FW - HD - H

  def body(p0_ref, p1_ref, r_ref, b_ref, w_ref, a_ref, bb_ref,
           fx_ref, el_ref, er_ref):
    acc = p0_ref[...] + p1_ref[...]
    esum = acc[:, 128:136]
    recip = jnp.where(esum > 0, 1.0 / esum, 0.0)
    scale = jnp.dot(recip, r_ref[...], preferred_element_type=jnp.float32)
    h = jnp.maximum(acc[:, 0:128] * scale + b_ref[...], 0.0)
    feat = jnp.dot(h, w_ref[...], preferred_element_type=jnp.float32)
    el = jnp.dot(feat, a_ref[...], preferred_element_type=jnp.float32)
    er = jnp.dot(feat, bb_ref[...], preferred_element_type=jnp.float32)
    z = jnp.zeros((BN, pad), jnp.float32)
    fx_ref[...] = jnp.concatenate([feat, el, z], axis=1)
    el_ref[...] = el
    er_ref[...] = jnp.concatenate(
        [er, jnp.zeros((BN, 16 - H), jnp.float32)], axis=1)

  return pl.pallas_call(
      body,
      grid=(GRID,),
      in_specs=[_blk(144), _blk(144), _full(R.shape), _full(bvec.shape),
                _full(W.shape), _full(Ablk.shape), _full(Bblk.shape)],
      out_specs=[_blk(FW), _blk(H), _blk(16)],
      out_shape=[
          jax.ShapeDtypeStruct((N, FW), jnp.float32),
          jax.ShapeDtypeStruct((N, H), jnp.float32),
          jax.ShapeDtypeStruct((N, 16), jnp.float32),
      ],
  )(p0, p1, R, bvec, W, Ablk, Bblk)


def _tc_final(q0, q1, b2vec):
  def body(q0_ref, q1_ref, b_ref, o_ref):
    acc = q0_ref[...] + q1_ref[...]
    esum = acc[:, 16:17]
    recip = jnp.where(esum > 0, 1.0 / esum, 0.0)
    o_ref[...] = acc[:, 0:16] * recip + b_ref[...]

  return pl.pallas_call(
      body,
      grid=(GRID,),
      in_specs=[_blk(32), _blk(32), _full(b2vec.shape)],
      out_specs=_blk(16),
      out_shape=jax.ShapeDtypeStruct((N, 16), jnp.float32),
  )(q0, q1, b2vec)


def _head_proj(a):
  H = a.shape[0]
  return (a[:, :, None] * jnp.eye(H, dtype=a.dtype)[:, None, :]).reshape(
      H * 16, H)


def _cvec(el, er16, H):
  c = jnp.maximum(jnp.max(el, axis=0) + jnp.max(er16[:, :H], axis=0), 0.0)
  return jnp.tile(c, 16 // H)


@jax.jit
def _run(inputs, edge_index, W0, al0, ar0, b0, W1, al1, ar1, b1,
         W2, al2, ar2, b2):
  src = edge_index[0].reshape(E // K, K)
  dst = edge_index[1].reshape(E // K, K)
  R = jnp.repeat(jnp.eye(8, dtype=jnp.float32), 16, axis=1)

  fx0, el0, er0 = _tc_pre(inputs, W0, _head_proj(al0), _head_proj(ar0), 144, 8)
  p = _sc_l01(fx0, er0, src, dst, _cvec(el0, er0, 8))

  fx1, el1, er1 = _tc_mid(p[0], p[1], R, b0.reshape(1, 128), W1,
                          _head_proj(al1), _head_proj(ar1), 144, 8)
  p = _sc_l01(fx1, er1, src, dst, _cvec(el1, er1, 8))

  fx2, el2, er2 = _tc_mid(p[0], p[1], R, b1.reshape(1, 128), W2,
                          _head_proj(al2), _head_proj(ar2), 32, 1)
  q = _sc_l2(fx2, er2, src, dst, _cvec(el2, er2, 1))

  return _tc_final(q[0], q[1], b2.reshape(1, 16))


def kernel(inputs, edge_index, W0, al0, ar0, b0, W1, al1, ar1, b1,
           W2, al2, ar2, b2):
  return _run(inputs, edge_index, W0, al0, ar0, b0, W1, al1, ar1, b1,
              W2, al2, ar2, b2)


# speedup vs baseline: 1.6308x; 1.1359x over previous
import functools

import jax
import jax.numpy as jnp
from jax import lax
from jax.experimental import pallas as pl
from jax.experimental.pallas import tpu as pltpu
from jax.experimental.pallas import tpu_sc as plsc

N = 10000
E = 320000
NC = 2
NS = 16
NW = NC * NS
EPW = E // NW
K = 80
UNROLL = 4
NCHUNK = EPW // K
SB = 5
CPW = NCHUNK
RPT = N // NS
RZB = 25


def _sc_edge_pass(H, FW):
  D = 16
  EL0 = H * D
  mesh = plsc.VectorSubcoreMesh(core_axis_name="c", subcore_axis_name="s")

  @functools.partial(
      pl.kernel,
      out_type=jax.ShapeDtypeStruct((NC, N, FW), jnp.float32),
      mesh=mesh,
      compiler_params=pltpu.CompilerParams(use_tc_tiling_on_sc=False),
      scratch_types=[
          pltpu.VMEM_SHARED((N, FW), jnp.float32),
          pltpu.VMEM((2, SB, K), jnp.int32),
          pltpu.VMEM((2, SB, K), jnp.int32),
          pltpu.VMEM((3, K, FW), jnp.float32),
          pltpu.VMEM((3, K, 16), jnp.float32),
          pltpu.VMEM((16,), jnp.float32),
          pltpu.SemaphoreType.DMA,
          pltpu.SemaphoreType.DMA,
          pltpu.SemaphoreType.DMA,
          pltpu.SemaphoreType.DMA,
          pltpu.SemaphoreType.DMA,
          pltpu.SemaphoreType.DMA,
          pltpu.SemaphoreType.DMA,
          pltpu.SemaphoreType.DMA,
          pltpu.SemaphoreType.DMA,
      ],
  )
  def edge_kernel(featx, er16, srcs, dsts, cvec, out, acc, sidx, didx,
                  rows, erb, cbuf, semr0, semr1, semr2, seme0, seme1, seme2,
                  sems0, sems1, sems2):
    cid = lax.axis_index("c")
    sid = lax.axis_index("s")
    wid = sid * NC + cid
    semr = (semr0, semr1, semr2)
    seme = (seme0, seme1, seme2)
    sems = (sems0, sems1, sems2)

    zero = jnp.zeros((16,), jnp.float32)

    def zrow(r, carry):
      for j in range(FW // 16):
        rows[0, r, pl.ds(j * 16, 16)] = zero
      return carry

    lax.fori_loop(0, K, zrow, 0)
    for j in range(RPT // K):
      pltpu.sync_copy(rows.at[0], acc.at[pl.ds(sid * RPT + j * K, K)])
    rem = RPT % K
    if rem:
      pltpu.sync_copy(rows.at[0, pl.ds(0, rem)],
                      acc.at[pl.ds(sid * RPT + (RPT // K) * K, rem)])
    pltpu.sync_copy(cvec, cbuf)
    plsc.subcore_barrier()

    cv = cbuf[...]
    mask = lax.iota(jnp.int32, 16) < H
    crow0 = wid * CPW

    def _idx_rows(ci):
      bb = lax.rem(ci // SB, 2)
      r = lax.rem(ci, SB)
      return sidx.at[bb, r], didx.at[bb, r]

    def start_gather(ci, b):
      @pl.when(lax.rem(ci, SB) == 0)
      def _():
        nb = ci // SB
        bb = lax.rem(nb, 2)
        pltpu.sync_copy(srcs.at[pl.ds(crow0 + nb * SB, SB)], sidx.at[bb])
        pltpu.sync_copy(dsts.at[pl.ds(crow0 + nb * SB, SB)], didx.at[bb])

      sr, dr = _idx_rows(ci)
      pltpu.make_async_copy(featx.at[sr], rows.at[b], semr[b]).start()
      pltpu.make_async_copy(er16.at[dr], erb.at[b], seme[b]).start()

    def wait_gather(ci, b):
      sr, dr = _idx_rows(ci)
      pltpu.make_async_copy(featx.at[sr], rows.at[b], semr[b]).wait()
      pltpu.make_async_copy(er16.at[dr], erb.at[b], seme[b]).wait()

    def process(ci, b):
      rb = rows.at[b]
      eb = erb.at[b]

      def edge(e):
        elv = rb[e, pl.ds(EL0, 16)]
        erv = eb[e]
        t = elv + erv
        t = jnp.maximum(t, 0.2 * t)
        ee = jnp.exp(t - cv)
        for h in range(H):
          idx = jnp.full((16,), h, jnp.int32)
          bc = jnp.take_along_axis(ee, idx, axis=0, mode="promise_in_bounds")
          rb[e, pl.ds(h * D, 16)] = rb[e, pl.ds(h * D, 16)] * bc
        rb[e, pl.ds(EL0, 16)] = jnp.where(mask, ee, 0.0)

      def edge4(i, c):
        for sub in range(UNROLL):
          edge(i * UNROLL + sub)
        return c

      lax.fori_loop(0, K // UNROLL, edge4, 0)

    def start_scatter(ci, b):
      _, dr = _idx_rows(ci)
      pltpu.make_async_copy(rows.at[b], acc.at[dr], sems[b]).start(add=True)

    def wait_scatter(ci, b):
      _, dr = _idx_rows(ci)
      pltpu.make_async_copy(rows.at[b], acc.at[dr], sems[b]).wait()

    def pos(c, j):
      b = j % 3
      b1 = (j + 1) % 3

      @pl.when(c >= 2)
      def _():
        wait_scatter(c - 2, b1)

      @pl.when(c + 1 < NCHUNK)
      def _():
        start_gather(c + 1, b1)

      wait_gather(c, b)
      process(c, b)
      start_scatter(c, b)

    start_gather(0, 0)

    def six(it, carry):
      c0 = it * 6
      for j in range(6):
        pos(c0 + j, j)
      return carry

    NFULL = NCHUNK // 6
    lax.fori_loop(0, NFULL, six, 0)
    for j in range(NCHUNK - NFULL * 6):
      pos(NFULL * 6 + j, j)
    wait_scatter(NCHUNK - 2, (NCHUNK - 2) % 3)
    wait_scatter(NCHUNK - 1, (NCHUNK - 1) % 3)
    plsc.subcore_barrier()
    r0 = sid * RPT
    pltpu.sync_copy(acc.at[pl.ds(r0, RPT)], out.at[cid, pl.ds(r0, RPT)])

  return edge_kernel


_sc_l01 = _sc_edge_pass(8, 144)
_sc_l2 = _sc_edge_pass(1, 32)

BN = 2000
GRID = N // BN


def _full(shape):
  return pl.BlockSpec(shape, lambda i: (0,) * len(shape))


def _blk(w):
  return pl.BlockSpec((BN, w), lambda i: (i, 0))


def _tc_pre(x, W, Ablk, Bblk, FW, H):
  HD = H * 16
  pad = FW - HD - H

  def body(x_ref, w_ref, a_ref, b_ref, fx_ref, el_ref, er_ref):
    feat = jnp.dot(x_ref[...], w_ref[...], preferred_element_type=jnp.float32)
    el = jnp.dot(feat, a_ref[...], preferred_element_type=jnp.float32)
    er = jnp.dot(feat, b_ref[...], preferred_element_type=jnp.float32)
    z = jnp.zeros((BN, pad), jnp.float32)
    fx_ref[...] = jnp.concatenate([feat, el, z], axis=1)
    el_ref[...] = el
    er_ref[...] = jnp.concatenate(
        [er, jnp.zeros((BN, 16 - H), jnp.float32)], axis=1)

  return pl.pallas_call(
      body,
      grid=(GRID,),
      in_specs=[_blk(x.shape[1]), _full(W.shape), _full(Ablk.shape),
                _full(Bblk.shape)],
      out_specs=[_blk(FW), _blk(H), _blk(16)],
      out_shape=[
          jax.ShapeDtypeStruct((N, FW), jnp.float32),
          jax.ShapeDtypeStruct((N, H), jnp.float32),
          jax.ShapeDtypeStruct((N, 16), jnp.float32),
      ],
  )(x, W, Ablk, Bblk)


def _tc_mid(p0, p1, R, bvec, W, Ablk, Bblk, FW, H):
  HD = H * 16
  pad = FW - HD - H

  def body(p0_ref, p1_ref, r_ref, b_ref, w_ref, a_ref, bb_ref,
           fx_ref, el_ref, er_ref):
    acc = p0_ref[...] + p1_ref[...]
    esum = acc[:, 128:136]
    recip = jnp.where(esum > 0, 1.0 / esum, 0.0)
    scale = jnp.dot(recip, r_ref[...], preferred_element_type=jnp.float32)
    h = jnp.maximum(acc[:, 0:128] * scale + b_ref[...], 0.0)
    feat = jnp.dot(h, w_ref[...], preferred_element_type=jnp.float32)
    el = jnp.dot(feat, a_ref[...], preferred_element_type=jnp.float32)
    er = jnp.dot(feat, bb_ref[...], preferred_element_type=jnp.float32)
    z = jnp.zeros((BN, pad), jnp.float32)
    fx_ref[...] = jnp.concatenate([feat, el, z], axis=1)
    el_ref[...] = el
    er_ref[...] = jnp.concatenate(
        [er, jnp.zeros((BN, 16 - H), jnp.float32)], axis=1)

  return pl.pallas_call(
      body,
      grid=(GRID,),
      in_specs=[_blk(144), _blk(144), _full(R.shape), _full(bvec.shape),
                _full(W.shape), _full(Ablk.shape), _full(Bblk.shape)],
      out_specs=[_blk(FW), _blk(H), _blk(16)],
      out_shape=[
          jax.ShapeDtypeStruct((N, FW), jnp.float32),
          jax.ShapeDtypeStruct((N, H), jnp.float32),
          jax.ShapeDtypeStruct((N, 16), jnp.float32),
      ],
  )(p0, p1, R, bvec, W, Ablk, Bblk)


def _tc_final(q0, q1, b2vec):
  def body(q0_ref, q1_ref, b_ref, o_ref):
    acc = q0_ref[...] + q1_ref[...]
    esum = acc[:, 16:17]
    recip = jnp.where(esum > 0, 1.0 / esum, 0.0)
    o_ref[...] = acc[:, 0:16] * recip + b_ref[...]

  return pl.pallas_call(
      body,
      grid=(GRID,),
      in_specs=[_blk(32), _blk(32), _full(b2vec.shape)],
      out_specs=_blk(16),
      out_shape=jax.ShapeDtypeStruct((N, 16), jnp.float32),
  )(q0, q1, b2vec)


def _head_proj(a):
  H = a.shape[0]
  return (a[:, :, None] * jnp.eye(H, dtype=a.dtype)[:, None, :]).reshape(
      H * 16, H)


def _cvec(el, er16, H):
  c = jnp.maximum(jnp.max(el, axis=0) + jnp.max(er16[:, :H], axis=0), 0.0)
  return jnp.tile(c, 16 // H)


@jax.jit
def _run(inputs, edge_index, W0, al0, ar0, b0, W1, al1, ar1, b1,
         W2, al2, ar2, b2):
  src = edge_index[0].reshape(E // K, K)
  dst = edge_index[1].reshape(E // K, K)
  R = jnp.repeat(jnp.eye(8, dtype=jnp.float32), 16, axis=1)

  fx0, el0, er0 = _tc_pre(inputs, W0, _head_proj(al0), _head_proj(ar0), 144, 8)
  p = _sc_l01(fx0, er0, src, dst, _cvec(el0, er0, 8))

  fx1, el1, er1 = _tc_mid(p[0], p[1], R, b0.reshape(1, 128), W1,
                          _head_proj(al1), _head_proj(ar1), 144, 8)
  p = _sc_l01(fx1, er1, src, dst, _cvec(el1, er1, 8))

  fx2, el2, er2 = _tc_mid(p[0], p[1], R, b1.reshape(1, 128), W2,
                          _head_proj(al2), _head_proj(ar2), 32, 1)
  q = _sc_l2(fx2, er2, src, dst, _cvec(el2, er2, 1))

  return _tc_final(q[0], q[1], b2.reshape(1, 16))


def kernel(inputs, edge_index, W0, al0, ar0, b0, W1, al1, ar1, b1,
           W2, al2, ar2, b2):
  return _run(inputs, edge_index, W0, al0, ar0, b0, W1, al1, ar1, b1,
              W2, al2, ar2, b2)
